# baseline (speedup 1.0000x reference)
import functools

import jax
import jax.numpy as jnp
from jax import lax
from jax.experimental import pallas as pl
from jax.experimental.pallas import tpu as pltpu
from jax.experimental.pallas import tpu_sc as plsc

VOCAB = 100000
D = 128
NLANE = 16
ND = D // NLANE
CHUNK = 128


def _ln_body(ids_hbm, tids_hbm, wemb_hbm, temb_hbm, gamma_hbm, beta_hbm,
             out_hbm, idx_v, tid_v, rows_v, temb_v, gamma_v, beta_v,
             gsems, ssem, osem, *, tokens_per_worker, num_cores):
    c = lax.axis_index("c")
    s = lax.axis_index("s")
    wid = s * num_cores + c
    base = wid * tokens_per_worker
    nchunk = tokens_per_worker // CHUNK
    seq = ids_hbm.shape[1]
    wpr = seq // tokens_per_worker
    row = wid // wpr
    col = (wid % wpr) * tokens_per_worker

    h_idx = pltpu.async_copy(ids_hbm.at[row, pl.ds(col, tokens_per_worker)],
                             idx_v, ssem)
    h_tid = pltpu.async_copy(tids_hbm.at[row, pl.ds(col, tokens_per_worker)],
                             tid_v, ssem)
    h_tab = pltpu.async_copy(temb_hbm, temb_v, ssem)
    h_g = pltpu.async_copy(gamma_hbm, gamma_v, ssem)
    h_b = pltpu.async_copy(beta_hbm, beta_v, ssem)
    h_idx.wait()

    gh = []
    for cc in range(nchunk):
        gh.append(pltpu.async_copy(
            wemb_hbm.at[idx_v.at[pl.ds(cc * CHUNK, CHUNK)]],
            rows_v.at[pl.ds(cc * CHUNK, CHUNK)],
            gsems[cc]))
    h_tid.wait()
    h_tab.wait()
    h_g.wait()
    h_b.wait()

    iota = lax.iota(jnp.int32, NLANE)
    tcols = [iota + d * NLANE for d in range(ND)]
    g = [gamma_v[pl.ds(d * NLANE, NLANE)] for d in range(ND)]
    b = [beta_v[pl.ds(d * NLANE, NLANE)] for d in range(ND)]

    oh = []
    for cc in range(nchunk):
        gh[cc].wait()

        @plsc.parallel_loop(cc * CHUNK, (cc + 1) * CHUNK, unroll=4)
        def tok(r):
            tvec = plsc.load_gather(tid_v, [jnp.broadcast_to(r, (NLANE,))])
            x = []
            ssum = None
            psum = None
            for d in range(ND):
                w = rows_v[r, pl.ds(d * NLANE, NLANE)]
                t = plsc.load_gather(temb_v, [tvec, tcols[d]])
                xd = w + t
                x.append(xd)
                ssum = xd if ssum is None else ssum + xd
                psum = xd * xd if psum is None else psum + xd * xd
            mu = jnp.sum(ssum) * (1.0 / D)
            var = jnp.sum(psum) * (1.0 / D) - mu * mu
            muv = jnp.broadcast_to(mu, (NLANE,))
            vv = jnp.broadcast_to(var + 1e-12, (NLANE,))
            bits = plsc.bitcast(vv, jnp.int32)
            y = plsc.bitcast(jnp.int32(0x5F3759DF) - (bits >> 1), jnp.float32)
            for _ in range(2):
                y = y * (1.5 - 0.5 * vv * y * y)
            for d in range(ND):
                rows_v[r, pl.ds(d * NLANE, NLANE)] = \
                    (x[d] - muv) * (y * g[d]) + b[d]

        oh.append(pltpu.async_copy(
            rows_v.at[pl.ds(cc * CHUNK, CHUNK)],
            out_hbm.at[pl.ds(base + cc * CHUNK, CHUNK)],
            osem))
    for h in oh:
        h.wait()


@jax.jit
def _emb_ln(ids, tids, word_emb, type_emb, gamma, beta):
    n_tokens = ids.shape[0] * ids.shape[1]
    info = plsc.get_sparse_core_info()
    nc, ns = info.num_cores, info.num_subcores
    tokens_per_worker = n_tokens // (nc * ns)
    mesh = plsc.VectorSubcoreMesh(core_axis_name="c", subcore_axis_name="s")
    run = functools.partial(
        pl.kernel,
        mesh=mesh,
        compiler_params=pltpu.CompilerParams(needs_layout_passes=False),
        out_type=jax.ShapeDtypeStruct((n_tokens, D), jnp.float32),
        scratch_types=[
            pltpu.VMEM((tokens_per_worker,), jnp.int32),
            pltpu.VMEM((tokens_per_worker,), jnp.int32),
            pltpu.VMEM((tokens_per_worker, D), jnp.float32),
            pltpu.VMEM((2, D), jnp.float32),
            pltpu.VMEM((D,), jnp.float32),
            pltpu.VMEM((D,), jnp.float32),
            [pltpu.SemaphoreType.DMA] * (tokens_per_worker // CHUNK),
            pltpu.SemaphoreType.DMA,
            pltpu.SemaphoreType.DMA,
        ],
    )(functools.partial(_ln_body,
                        tokens_per_worker=tokens_per_worker,
                        num_cores=nc))
    return run(ids, tids, word_emb, type_emb, gamma, beta)


def kernel(input_ids, token_type_ids, word_emb, type_emb, gamma, beta):
    bsz, seq = input_ids.shape
    out = _emb_ln(input_ids.astype(jnp.int32), token_type_ids.astype(jnp.int32),
                  word_emb, type_emb, gamma, beta)
    return out.reshape(bsz, seq, D)


# speedup vs baseline: 1.4092x; 1.4092x over previous
import functools

import jax
import jax.numpy as jnp
from jax import lax
from jax.experimental import pallas as pl
from jax.experimental.pallas import tpu as pltpu
from jax.experimental.pallas import tpu_sc as plsc

VOCAB = 100000
D = 128
NLANE = 16
ND = D // NLANE
CHUNK = 128


def _ln_body(ids_hbm, tids_hbm, wemb_hbm, temb_hbm, gamma_hbm, beta_hbm,
             out_hbm, idx_v, tid_v, rows_v, temb_v, gamma_v, beta_v,
             gsems, ssem, osem, *, tokens_per_worker, num_cores):
    c = lax.axis_index("c")
    s = lax.axis_index("s")
    wid = s * num_cores + c
    base = wid * tokens_per_worker
    nchunk = tokens_per_worker // CHUNK
    seq = ids_hbm.shape[1]
    wpr = seq // tokens_per_worker
    row = wid // wpr
    col = (wid % wpr) * tokens_per_worker

    h_idx = pltpu.async_copy(ids_hbm.at[row, pl.ds(col, tokens_per_worker)],
                             idx_v, ssem)
    h_tid = pltpu.async_copy(tids_hbm.at[row, pl.ds(col, tokens_per_worker)],
                             tid_v, ssem)
    h_tab = pltpu.async_copy(temb_hbm, temb_v, ssem)
    h_g = pltpu.async_copy(gamma_hbm, gamma_v, ssem)
    h_b = pltpu.async_copy(beta_hbm, beta_v, ssem)
    h_idx.wait()

    gh = []
    for cc in range(nchunk):
        gh.append(pltpu.async_copy(
            wemb_hbm.at[idx_v.at[pl.ds(cc * CHUNK, CHUNK)]],
            rows_v.at[pl.ds(cc * CHUNK, CHUNK)],
            gsems[cc]))
    h_tid.wait()
    h_tab.wait()
    h_g.wait()
    h_b.wait()

    iota = lax.iota(jnp.int32, NLANE)
    tcols = [iota + d * NLANE for d in range(ND)]
    g = [gamma_v[pl.ds(d * NLANE, NLANE)] for d in range(ND)]
    b = [beta_v[pl.ds(d * NLANE, NLANE)] for d in range(ND)]

    oh = []
    for cc in range(nchunk):
        gh[cc].wait()

        @plsc.parallel_loop(cc * CHUNK, (cc + 1) * CHUNK, unroll=2)
        def tok(r):
            tvec = plsc.load_gather(tid_v, [jnp.broadcast_to(r, (NLANE,))])
            x = []
            ssum = None
            psum = None
            for d in range(ND):
                w = rows_v[r, pl.ds(d * NLANE, NLANE)]
                t = plsc.load_gather(temb_v, [tvec, tcols[d]])
                xd = w + t
                x.append(xd)
                ssum = xd if ssum is None else ssum + xd
                psum = xd * xd if psum is None else psum + xd * xd
            mu = jnp.sum(ssum) * (1.0 / D)
            var = jnp.sum(psum) * (1.0 / D) - mu * mu
            muv = jnp.broadcast_to(mu, (NLANE,))
            vv = jnp.broadcast_to(var + 1e-12, (NLANE,))
            bits = plsc.bitcast(vv, jnp.int32)
            y = plsc.bitcast(jnp.int32(0x5F3759DF) - (bits >> 1), jnp.float32)
            for _ in range(2):
                y = y * (1.5 - 0.5 * vv * y * y)
            for d in range(ND):
                rows_v[r, pl.ds(d * NLANE, NLANE)] = \
                    (x[d] - muv) * (y * g[d]) + b[d]

        oh.append(pltpu.async_copy(
            rows_v.at[pl.ds(cc * CHUNK, CHUNK)],
            out_hbm.at[pl.ds(base + cc * CHUNK, CHUNK)],
            osem))
    for h in oh:
        h.wait()


@jax.jit
def _emb_ln(ids, tids, word_emb, type_emb, gamma, beta):
    n_tokens = ids.shape[0] * ids.shape[1]
    info = plsc.get_sparse_core_info()
    nc, ns = info.num_cores, info.num_subcores
    tokens_per_worker = n_tokens // (nc * ns)
    mesh = plsc.VectorSubcoreMesh(core_axis_name="c", subcore_axis_name="s")
    run = functools.partial(
        pl.kernel,
        mesh=mesh,
        compiler_params=pltpu.CompilerParams(needs_layout_passes=False),
        out_type=jax.ShapeDtypeStruct((n_tokens, D), jnp.float32),
        scratch_types=[
            pltpu.VMEM((tokens_per_worker,), jnp.int32),
            pltpu.VMEM((tokens_per_worker,), jnp.int32),
            pltpu.VMEM((tokens_per_worker, D), jnp.float32),
            pltpu.VMEM((2, D), jnp.float32),
            pltpu.VMEM((D,), jnp.float32),
            pltpu.VMEM((D,), jnp.float32),
            [pltpu.SemaphoreType.DMA] * (tokens_per_worker // CHUNK),
            pltpu.SemaphoreType.DMA,
            pltpu.SemaphoreType.DMA,
        ],
    )(functools.partial(_ln_body,
                        tokens_per_worker=tokens_per_worker,
                        num_cores=nc))
    return run(ids, tids, word_emb, type_emb, gamma, beta)


def kernel(input_ids, token_type_ids, word_emb, type_emb, gamma, beta):
    bsz, seq = input_ids.shape
    out = _emb_ln(input_ids.astype(jnp.int32), token_type_ids.astype(jnp.int32),
                  word_emb, type_emb, gamma, beta)
    return out.reshape(bsz, seq, D)
